# Initial kernel scaffold; baseline (speedup 1.0000x reference)
import jax
import jax.numpy as jnp
from jax.experimental import pallas as pl


def kernel(x, edge_index, W_self0, W_neigh0, b0, W_self1, W_neigh1, b1, W_self2, W_neigh2, b2, gamma0, beta0, gamma1, beta1):
    raise NotImplementedError("write your pallas kernel here")


# speedup vs baseline: 4.4307x; 4.4307x over previous
import functools

import jax
import jax.numpy as jnp
from jax import lax
from jax.experimental import pallas as pl
from jax.experimental.pallas import tpu as pltpu
from jax.experimental.pallas import tpu_sc as plsc

N = 10000
E = 320000
NP = 10240
B = 128
S1 = -(-E // (32 * B))
RPT = NP // 16
ZR = 64
R = 512
NG = NP // R

_f32 = jnp.float32
_i32 = jnp.int32


def _zero_zbuf(zbuf, ncols):
    zero16 = jnp.zeros((16,), _f32)

    def zrow(r, carry):
        for k in range(ncols // 16):
            zbuf[r, pl.ds(k * 16, 16)] = zero16
        return carry

    lax.fori_loop(0, ZR, zrow, 0)


def _zero_acc(zbuf, acc, row0):
    def zchunk(i, carry):
        pltpu.sync_copy(zbuf, acc.at[pl.ds(row0 + i * ZR, ZR)])
        return carry

    lax.fori_loop(0, RPT // ZR, zchunk, 0)


def _make_sc_edgesplit(F, with_deg):
    mesh = plsc.VectorSubcoreMesh(core_axis_name="c", subcore_axis_name="s")
    out_type = [jax.ShapeDtypeStruct((2, NP, F), _f32)]
    scratch = [
        pltpu.VMEM((S1, B), _i32),
        pltpu.VMEM((S1, B), _i32),
        pltpu.VMEM((B, F), _f32),
        pltpu.VMEM((ZR, F), _f32),
        pltpu.VMEM_SHARED((NP, F), _f32),
        pltpu.SemaphoreType.DMA,
    ]
    if with_deg:
        out_type.append(jax.ShapeDtypeStruct((2, NP), _f32))
        scratch += [
            pltpu.VMEM((B,), _f32),
            pltpu.VMEM((RPT,), _f32),
            pltpu.VMEM_SHARED((NP,), _f32),
        ]

    @functools.partial(pl.kernel, out_type=out_type, mesh=mesh,
                       scratch_types=scratch)
    def k(table, srcs, dsts, *refs):
        if with_deg:
            (out, deg, src_v, dst_v, rows_v, zbuf, acc, sem,
             ones_v, zd, accd) = refs
        else:
            out, src_v, dst_v, rows_v, zbuf, acc, sem = refs
        c = lax.axis_index("c")
        s = lax.axis_index("s")
        wid = c * 16 + s
        row0 = s * RPT

        pltpu.sync_copy(srcs.at[wid], src_v)
        pltpu.sync_copy(dsts.at[wid], dst_v)
        _zero_zbuf(zbuf, F)
        _zero_acc(zbuf, acc, row0)
        if with_deg:
            one16 = jnp.ones((16,), _f32)
            zero16 = jnp.zeros((16,), _f32)
            for kk in range(B // 16):
                ones_v[pl.ds(kk * 16, 16)] = one16
            for kk in range(RPT // 16):
                zd[pl.ds(kk * 16, 16)] = zero16
            pltpu.sync_copy(zd, accd.at[pl.ds(row0, RPT)])
        plsc.subcore_barrier()

        def step(j, carry):
            pltpu.async_copy(table.at[src_v.at[j]], rows_v, sem).wait()
            pltpu.sync_copy(rows_v, acc.at[dst_v.at[j]], add=True)
            if with_deg:
                pltpu.sync_copy(ones_v, accd.at[dst_v.at[j]], add=True)
            return carry

        lax.fori_loop(0, S1, step, 0)
        plsc.subcore_barrier()

        pltpu.sync_copy(acc.at[pl.ds(row0, RPT)],
                        out.at[c, pl.ds(row0, RPT)])
        if with_deg:
            pltpu.sync_copy(accd.at[pl.ds(row0, RPT)],
                            deg.at[c, pl.ds(row0, RPT)])

    return k


def _make_sc_colblock():
    F = 128
    mesh = plsc.VectorSubcoreMesh(core_axis_name="c", subcore_axis_name="s")
    out_type = [jax.ShapeDtypeStruct((2, NP, F), _f32)]
    scratch = [
        pltpu.VMEM((S1, B), _i32),
        pltpu.VMEM((S1, B), _i32),
        pltpu.VMEM((B, F), _f32),
        pltpu.VMEM((ZR, F), _f32),
        pltpu.VMEM_SHARED((NP, F), _f32),
        pltpu.SemaphoreType.DMA,
    ]

    @functools.partial(pl.kernel, out_type=out_type, mesh=mesh,
                       scratch_types=scratch)
    def k(table, srcs, dsts, out, src_v, dst_v, rows_v, zbuf, acc, sem):
        c = lax.axis_index("c")
        s = lax.axis_index("s")
        row0 = s * RPT

        pltpu.sync_copy(srcs.at[s], src_v)
        pltpu.sync_copy(dsts.at[s], dst_v)

        off = c * NP

        def offrow(j, carry):
            for kk in range(B // 16):
                src_v[j, pl.ds(kk * 16, 16)] = (
                    src_v[j, pl.ds(kk * 16, 16)] + off)
            return carry

        lax.fori_loop(0, S1, offrow, 0)

        _zero_zbuf(zbuf, F)
        _zero_acc(zbuf, acc, row0)
        plsc.subcore_barrier()

        def step(j, carry):
            pltpu.async_copy(table.at[src_v.at[j]], rows_v, sem).wait()
            pltpu.sync_copy(rows_v, acc.at[dst_v.at[j]], add=True)
            return carry

        lax.fori_loop(0, S1, step, 0)
        plsc.subcore_barrier()

        pltpu.sync_copy(acc.at[pl.ds(row0, RPT)],
                        out.at[c, pl.ds(row0, RPT)])

    return k


def _inv_deg(da, db):
    return 1.0 / jnp.maximum(da[...] + db[...], 1.0)


def _accum_stats(i, zv, st):
    rows = lax.broadcasted_iota(_i32, (R, 1), 0) + i * R
    m = (rows < N).astype(_f32)
    zm = zv * m
    s1 = jnp.sum(zm, axis=0, keepdims=True)
    s2 = jnp.sum(zm * zm, axis=0, keepdims=True)
    upd = jnp.concatenate(
        [s1, s2, jnp.zeros((6, s1.shape[1]), _f32)], axis=0)

    @pl.when(i == 0)
    def _():
        st[...] = upd

    @pl.when(i != 0)
    def _():
        st[...] = st[...] + upd


def _dense0_body(x, agg, da, db, Ws, Wn, b, z, st):
    i = pl.program_id(0)
    inv = _inv_deg(da, db)
    hn = (agg[0] + agg[1]) * inv
    zv = (jnp.dot(x[...], Ws[...], preferred_element_type=_f32)
          + jnp.dot(hn, Wn[...], preferred_element_type=_f32) + b[...])
    z[...] = zv
    _accum_stats(i, zv, st)


def _dense1_body(h, p1, p2, da, db, Ws, Wn, b, z, st):
    i = pl.program_id(0)
    inv = _inv_deg(da, db)
    a0 = (p1[0] + p2[0]) * inv
    a1 = (p1[1] + p2[1]) * inv
    zv = (jnp.dot(h[0], Ws[0:128, :], preferred_element_type=_f32)
          + jnp.dot(h[1], Ws[128:256, :], preferred_element_type=_f32)
          + jnp.dot(a0, Wn[0:128, :], preferred_element_type=_f32)
          + jnp.dot(a1, Wn[128:256, :], preferred_element_type=_f32)
          + b[...])
    z[...] = zv
    _accum_stats(i, zv, st)


def _norm_body(z, st, gamma, beta, h):
    mu = st[0:1, :] * (1.0 / N)
    ms = st[1:2, :] * (1.0 / N)
    rstd = lax.rsqrt(ms - mu * mu + 1e-5)
    hv = jnp.maximum((z[...] - mu) * rstd * gamma[...] + beta[...], 0.0)
    h[0] = hv[:, 0:128]
    h[1] = hv[:, 128:256]


def _proj2_body(h, Wn, Ws, b, p2, t2):
    p2[...] = (jnp.dot(h[0], Wn[0:128, :], preferred_element_type=_f32)
               + jnp.dot(h[1], Wn[128:256, :], preferred_element_type=_f32))
    t2[...] = (jnp.dot(h[0], Ws[0:128, :], preferred_element_type=_f32)
               + jnp.dot(h[1], Ws[128:256, :], preferred_element_type=_f32)
               + b[...])


def _final_body(t2, q, da, db, o):
    o[...] = t2[...] + (q[0] + q[1]) * _inv_deg(da, db)


def _row_spec(w):
    return pl.BlockSpec((R, w), lambda i: (i, 0))


def _row2_spec(w):
    return pl.BlockSpec((2, R, w), lambda i: (0, i, 0))


def _whole_spec(shape):
    nd = len(shape)
    return pl.BlockSpec(shape, lambda i: (0,) * nd)


_ARB = pltpu.CompilerParams(dimension_semantics=("arbitrary",))


def _tc_call(body, in_specs, out_specs, out_shape):
    return pl.pallas_call(
        body, grid=(NG,), in_specs=in_specs, out_specs=out_specs,
        out_shape=out_shape, compiler_params=_ARB)


_dense0 = _tc_call(
    _dense0_body,
    [_row_spec(128), _row2_spec(128), _row_spec(1), _row_spec(1),
     _whole_spec((128, 256)), _whole_spec((128, 256)), _whole_spec((1, 256))],
    [_row_spec(256), _whole_spec((8, 256))],
    [jax.ShapeDtypeStruct((NP, 256), _f32),
     jax.ShapeDtypeStruct((8, 256), _f32)])

_dense1 = _tc_call(
    _dense1_body,
    [_row2_spec(128), _row2_spec(128), _row2_spec(128),
     _row_spec(1), _row_spec(1),
     _whole_spec((256, 256)), _whole_spec((256, 256)), _whole_spec((1, 256))],
    [_row_spec(256), _whole_spec((8, 256))],
    [jax.ShapeDtypeStruct((NP, 256), _f32),
     jax.ShapeDtypeStruct((8, 256), _f32)])

_norm = _tc_call(
    _norm_body,
    [_row_spec(256), _whole_spec((8, 256)),
     _whole_spec((1, 256)), _whole_spec((1, 256))],
    [_row2_spec(128)],
    [jax.ShapeDtypeStruct((2, NP, 128), _f32)])

_proj2 = _tc_call(
    _proj2_body,
    [_row2_spec(128),
     _whole_spec((256, 128)), _whole_spec((256, 128)), _whole_spec((1, 128))],
    [_row_spec(128), _row_spec(128)],
    [jax.ShapeDtypeStruct((NP, 128), _f32),
     jax.ShapeDtypeStruct((NP, 128), _f32)])

_final = _tc_call(
    _final_body,
    [_row_spec(128), _row2_spec(128), _row_spec(1), _row_spec(1)],
    [_row_spec(128)],
    [jax.ShapeDtypeStruct((NP, 128), _f32)])


_sc_edge128 = _make_sc_edgesplit(128, with_deg=True)
_sc_edge64 = _make_sc_edgesplit(128, with_deg=False)
_sc_colblock = _make_sc_colblock()

_EH = 16 * S1 * B


def _pad_edges(s, d, n_groups):
    tot = n_groups * S1 * B
    pad = tot - s.shape[0]
    sp = jnp.concatenate([s, jnp.zeros((pad,), _i32)]).reshape(n_groups, S1, B)
    dp = jnp.concatenate([d, jnp.full((pad,), N, _i32)]).reshape(n_groups, S1, B)
    return sp, dp


def kernel(x, edge_index, W_self0, W_neigh0, b0, W_self1, W_neigh1, b1,
           W_self2, W_neigh2, b2, gamma0, beta0, gamma1, beta1):
    src = edge_index[0].astype(_i32)
    dst = edge_index[1].astype(_i32)

    srcES, dstES = _pad_edges(src, dst, 32)
    srcH1, dstH1 = _pad_edges(src[:E // 2], dst[:E // 2], 16)
    srcH2, dstH2 = _pad_edges(src[E // 2:], dst[E // 2:], 16)

    xp = jnp.pad(x, ((0, NP - N), (0, 0)))

    agg0, deg = _sc_edge128(x, srcES, dstES)
    da = deg[0].reshape(NP, 1)
    db = deg[1].reshape(NP, 1)
    z0, st0 = _dense0(xp, agg0, da, db, W_self0, W_neigh0, b0.reshape(1, -1))
    (h0,) = _norm(z0, st0, gamma0.reshape(1, -1), beta0.reshape(1, -1))

    h0s = h0.reshape(2 * NP, 128)
    (p1,) = _sc_colblock(h0s, srcH1, dstH1)
    (p2,) = _sc_colblock(h0s, srcH2, dstH2)
    z1, st1 = _dense1(h0, p1, p2, da, db, W_self1, W_neigh1,
                      b1.reshape(1, -1))
    (h1,) = _norm(z1, st1, gamma1.reshape(1, -1), beta1.reshape(1, -1))

    Wn2p = jnp.pad(W_neigh2, ((0, 0), (0, 88)))
    Ws2p = jnp.pad(W_self2, ((0, 0), (0, 88)))
    b2p = jnp.pad(b2, (0, 88)).reshape(1, -1)
    pr2, t2 = _proj2(h1, Wn2p, Ws2p, b2p)
    (q2,) = _sc_edge64(pr2, srcES, dstES)
    (o,) = _final(t2, q2, da, db)
    return o[:N, :40]


# speedup vs baseline: 5.3139x; 1.1993x over previous
import functools

import jax
import jax.numpy as jnp
from jax import lax
from jax.experimental import pallas as pl
from jax.experimental.pallas import tpu as pltpu
from jax.experimental.pallas import tpu_sc as plsc

N = 10000
E = 320000
NP = 10240
B = 128
S1 = 79
RPT = NP // 16
R = 512
NG = NP // R

_f32 = jnp.float32
_i32 = jnp.int32


def _zero_rows(rows, ncols):
    zero16 = jnp.zeros((16,), _f32)

    def zrow(r, carry):
        for k in range(ncols // 16):
            rows[r, pl.ds(k * 16, 16)] = zero16
        return carry

    lax.fori_loop(0, B, zrow, 0)


def _zero_acc(rows, acc, row0):
    def zchunk(i, carry):
        pltpu.sync_copy(rows, acc.at[pl.ds(row0 + i * B, B)])
        return carry

    lax.fori_loop(0, RPT // B, zchunk, 0)


def _make_sc_edgesplit(F, with_deg):
    mesh = plsc.VectorSubcoreMesh(core_axis_name="c", subcore_axis_name="s")
    out_type = [jax.ShapeDtypeStruct((2, NP, F), _f32)]
    scratch = [
        pltpu.VMEM((S1, B), _i32),
        pltpu.VMEM((2, B), _i32),
        pltpu.VMEM((B, F), _f32),
        pltpu.VMEM((B, F), _f32),
        pltpu.VMEM_SHARED((NP, F), _f32),
        pltpu.SemaphoreType.DMA,
        pltpu.SemaphoreType.DMA,
        pltpu.SemaphoreType.DMA,
        pltpu.SemaphoreType.DMA,
    ]
    if with_deg:
        out_type.append(jax.ShapeDtypeStruct((2, NP), _f32))
        scratch += [
            pltpu.VMEM((B,), _f32),
            pltpu.VMEM((RPT,), _f32),
            pltpu.VMEM_SHARED((NP,), _f32),
        ]

    @functools.partial(pl.kernel, out_type=out_type, mesh=mesh,
                       scratch_types=scratch)
    def k(table, srcs, dsts, *refs):
        if with_deg:
            (out, deg, src_v, dst_v, rows0, rows1, acc,
             gs0, gs1, ds0, ds1, ones_v, zd, accd) = refs
        else:
            (out, src_v, dst_v, rows0, rows1, acc,
             gs0, gs1, ds0, ds1) = refs
        rows = (rows0, rows1)
        gsem = (gs0, gs1)
        dsem = (ds0, ds1)
        c = lax.axis_index("c")
        s = lax.axis_index("s")
        wid = c * 16 + s
        row0 = s * RPT

        pltpu.sync_copy(srcs.at[wid], src_v)
        _zero_rows(rows0, F)
        _zero_acc(rows0, acc, row0)
        if with_deg:
            one16 = jnp.ones((16,), _f32)
            zero16 = jnp.zeros((16,), _f32)
            for kk in range(B // 16):
                ones_v[pl.ds(kk * 16, 16)] = one16
            for kk in range(RPT // 16):
                zd[pl.ds(kk * 16, 16)] = zero16
            pltpu.sync_copy(zd, accd.at[pl.ds(row0, RPT)])
        plsc.subcore_barrier()

        dbase = wid * S1
        for b in range(2):
            pltpu.async_copy(table.at[src_v.at[b]], rows[b], gsem[b])
            pltpu.async_copy(dsts.at[dbase + b], dst_v.at[b], dsem[b])

        def emit(j, b, issue):
            pltpu.make_async_copy(
                table.at[pl.ds(0, B)], rows[b], gsem[b]).wait()
            pltpu.make_async_copy(
                dsts.at[dbase], dst_v.at[b], dsem[b]).wait()
            pltpu.sync_copy(rows[b], acc.at[dst_v.at[b]], add=True)
            if with_deg:
                pltpu.sync_copy(ones_v, accd.at[dst_v.at[b]], add=True)
            if issue:
                pltpu.async_copy(table.at[src_v.at[j + 2]], rows[b], gsem[b])
                pltpu.async_copy(dsts.at[dbase + j + 2], dst_v.at[b], dsem[b])

        def step(i, carry):
            g = i * 2
            for b in range(2):
                emit(g + b, b, True)
            return carry

        lax.fori_loop(0, (S1 - 3) // 2, step, 0)
        emit(S1 - 3, 0, True)
        emit(S1 - 2, 1, False)
        emit(S1 - 1, 0, False)
        plsc.subcore_barrier()

        pltpu.sync_copy(acc.at[pl.ds(row0, RPT)],
                        out.at[c, pl.ds(row0, RPT)])
        if with_deg:
            pltpu.sync_copy(accd.at[pl.ds(row0, RPT)],
                            deg.at[c, pl.ds(row0, RPT)])

    return k


def _make_sc_colblock():
    F = 128
    mesh = plsc.VectorSubcoreMesh(core_axis_name="c", subcore_axis_name="s")
    out_type = [jax.ShapeDtypeStruct((2, NP, F), _f32)]
    scratch = [
        pltpu.VMEM((S1, B), _i32),
        pltpu.VMEM((2, B), _i32),
        pltpu.VMEM((B, F), _f32),
        pltpu.VMEM((B, F), _f32),
        pltpu.VMEM_SHARED((NP, F), _f32),
        pltpu.SemaphoreType.DMA,
        pltpu.SemaphoreType.DMA,
        pltpu.SemaphoreType.DMA,
        pltpu.SemaphoreType.DMA,
    ]

    @functools.partial(pl.kernel, out_type=out_type, mesh=mesh,
                       scratch_types=scratch)
    def k(table, srcs, dsts, out, src_v, dst_v, rows0, rows1, acc,
          gs0, gs1, ds0, ds1):
        rows = (rows0, rows1)
        gsem = (gs0, gs1)
        dsem = (ds0, ds1)
        c = lax.axis_index("c")
        s = lax.axis_index("s")
        row0 = s * RPT

        pltpu.sync_copy(srcs.at[s], src_v)

        off = c * NP

        def offrow(j, carry):
            for kk in range(B // 16):
                src_v[j, pl.ds(kk * 16, 16)] = (
                    src_v[j, pl.ds(kk * 16, 16)] + off)
            return carry

        lax.fori_loop(0, S1, offrow, 0)

        _zero_rows(rows0, F)
        _zero_acc(rows0, acc, row0)
        plsc.subcore_barrier()

        dbase = s * S1
        for b in range(2):
            pltpu.async_copy(table.at[src_v.at[b]], rows[b], gsem[b])
            pltpu.async_copy(dsts.at[dbase + b], dst_v.at[b], dsem[b])

        def emit(j, b, issue):
            pltpu.make_async_copy(
                table.at[pl.ds(0, B)], rows[b], gsem[b]).wait()
            pltpu.make_async_copy(
                dsts.at[dbase], dst_v.at[b], dsem[b]).wait()
            pltpu.sync_copy(rows[b], acc.at[dst_v.at[b]], add=True)
            if issue:
                pltpu.async_copy(table.at[src_v.at[j + 2]], rows[b], gsem[b])
                pltpu.async_copy(dsts.at[dbase + j + 2], dst_v.at[b], dsem[b])

        def step(i, carry):
            g = i * 2
            for b in range(2):
                emit(g + b, b, True)
            return carry

        lax.fori_loop(0, (S1 - 3) // 2, step, 0)
        emit(S1 - 3, 0, True)
        emit(S1 - 2, 1, False)
        emit(S1 - 1, 0, False)
        plsc.subcore_barrier()

        pltpu.sync_copy(acc.at[pl.ds(row0, RPT)],
                        out.at[c, pl.ds(row0, RPT)])

    return k


def _inv_deg(da, db):
    return 1.0 / jnp.maximum(da[...] + db[...], 1.0)


def _accum_stats(i, zv, st):
    rows = lax.broadcasted_iota(_i32, (R, 1), 0) + i * R
    m = (rows < N).astype(_f32)
    zm = zv * m
    s1 = jnp.sum(zm, axis=0, keepdims=True)
    s2 = jnp.sum(zm * zm, axis=0, keepdims=True)
    upd = jnp.concatenate(
        [s1, s2, jnp.zeros((6, s1.shape[1]), _f32)], axis=0)

    @pl.when(i == 0)
    def _():
        st[...] = upd

    @pl.when(i != 0)
    def _():
        st[...] = st[...] + upd


def _dense0_body(x, agg, da, db, Ws, Wn, b, z, st):
    i = pl.program_id(0)
    inv = _inv_deg(da, db)
    hn = (agg[0] + agg[1]) * inv
    zv = (jnp.dot(x[...], Ws[...], preferred_element_type=_f32)
          + jnp.dot(hn, Wn[...], preferred_element_type=_f32) + b[...])
    z[...] = zv
    _accum_stats(i, zv, st)


def _dense1_body(h, p1, p2, da, db, Ws, Wn, b, z, st):
    i = pl.program_id(0)
    inv = _inv_deg(da, db)
    a0 = (p1[0] + p2[0]) * inv
    a1 = (p1[1] + p2[1]) * inv
    zv = (jnp.dot(h[0], Ws[0:128, :], preferred_element_type=_f32)
          + jnp.dot(h[1], Ws[128:256, :], preferred_element_type=_f32)
          + jnp.dot(a0, Wn[0:128, :], preferred_element_type=_f32)
          + jnp.dot(a1, Wn[128:256, :], preferred_element_type=_f32)
          + b[...])
    z[...] = zv
    _accum_stats(i, zv, st)


def _norm_body(z, st, gamma, beta, h):
    mu = st[0:1, :] * (1.0 / N)
    ms = st[1:2, :] * (1.0 / N)
    rstd = lax.rsqrt(ms - mu * mu + 1e-5)
    hv = jnp.maximum((z[...] - mu) * rstd * gamma[...] + beta[...], 0.0)
    h[0] = hv[:, 0:128]
    h[1] = hv[:, 128:256]


def _proj2_body(h, Wn, Ws, b, p2, t2):
    p2[...] = (jnp.dot(h[0], Wn[0:128, :], preferred_element_type=_f32)
               + jnp.dot(h[1], Wn[128:256, :], preferred_element_type=_f32))
    t2[...] = (jnp.dot(h[0], Ws[0:128, :], preferred_element_type=_f32)
               + jnp.dot(h[1], Ws[128:256, :], preferred_element_type=_f32)
               + b[...])


def _final_body(t2, q, da, db, o):
    o[...] = t2[...] + (q[0] + q[1]) * _inv_deg(da, db)


def _row_spec(w):
    return pl.BlockSpec((R, w), lambda i: (i, 0))


def _row2_spec(w):
    return pl.BlockSpec((2, R, w), lambda i: (0, i, 0))


def _whole_spec(shape):
    nd = len(shape)
    return pl.BlockSpec(shape, lambda i: (0,) * nd)


_ARB = pltpu.CompilerParams(dimension_semantics=("arbitrary",))


def _tc_call(body, in_specs, out_specs, out_shape):
    return pl.pallas_call(
        body, grid=(NG,), in_specs=in_specs, out_specs=out_specs,
        out_shape=out_shape, compiler_params=_ARB)


_dense0 = _tc_call(
    _dense0_body,
    [_row_spec(128), _row2_spec(128), _row_spec(1), _row_spec(1),
     _whole_spec((128, 256)), _whole_spec((128, 256)), _whole_spec((1, 256))],
    [_row_spec(256), _whole_spec((8, 256))],
    [jax.ShapeDtypeStruct((NP, 256), _f32),
     jax.ShapeDtypeStruct((8, 256), _f32)])

_dense1 = _tc_call(
    _dense1_body,
    [_row2_spec(128), _row2_spec(128), _row2_spec(128),
     _row_spec(1), _row_spec(1),
     _whole_spec((256, 256)), _whole_spec((256, 256)), _whole_spec((1, 256))],
    [_row_spec(256), _whole_spec((8, 256))],
    [jax.ShapeDtypeStruct((NP, 256), _f32),
     jax.ShapeDtypeStruct((8, 256), _f32)])

_norm = _tc_call(
    _norm_body,
    [_row_spec(256), _whole_spec((8, 256)),
     _whole_spec((1, 256)), _whole_spec((1, 256))],
    [_row2_spec(128)],
    [jax.ShapeDtypeStruct((2, NP, 128), _f32)])

_proj2 = _tc_call(
    _proj2_body,
    [_row2_spec(128),
     _whole_spec((256, 128)), _whole_spec((256, 128)), _whole_spec((1, 128))],
    [_row_spec(128), _row_spec(128)],
    [jax.ShapeDtypeStruct((NP, 128), _f32),
     jax.ShapeDtypeStruct((NP, 128), _f32)])

_final = _tc_call(
    _final_body,
    [_row_spec(128), _row2_spec(128), _row_spec(1), _row_spec(1)],
    [_row_spec(128)],
    [jax.ShapeDtypeStruct((NP, 128), _f32)])


_sc_edge128 = _make_sc_edgesplit(128, with_deg=True)
_sc_edge64 = _make_sc_edgesplit(128, with_deg=False)
_sc_colblock = _make_sc_colblock()

_EH = 16 * S1 * B


def _pad_edges(s, d, n_groups):
    tot = n_groups * S1 * B
    pad = tot - s.shape[0]
    sp = jnp.concatenate([s, jnp.zeros((pad,), _i32)]).reshape(n_groups, S1, B)
    dp = jnp.concatenate([d, jnp.full((pad,), N, _i32)]).reshape(n_groups, S1, B)
    return sp, dp


def kernel(x, edge_index, W_self0, W_neigh0, b0, W_self1, W_neigh1, b1,
           W_self2, W_neigh2, b2, gamma0, beta0, gamma1, beta1):
    src = edge_index[0].astype(_i32)
    dst = edge_index[1].astype(_i32)

    srcES, dstES = _pad_edges(src, dst, 32)
    srcH1, dstH1 = _pad_edges(src[:E // 2], dst[:E // 2], 16)
    srcH2, dstH2 = _pad_edges(src[E // 2:], dst[E // 2:], 16)
    dstES = dstES.reshape(32 * S1, B)
    dstH1 = dstH1.reshape(16 * S1, B)
    dstH2 = dstH2.reshape(16 * S1, B)

    xp = jnp.pad(x, ((0, NP - N), (0, 0)))

    agg0, deg = _sc_edge128(x, srcES, dstES)
    da = deg[0].reshape(NP, 1)
    db = deg[1].reshape(NP, 1)
    z0, st0 = _dense0(xp, agg0, da, db, W_self0, W_neigh0, b0.reshape(1, -1))
    (h0,) = _norm(z0, st0, gamma0.reshape(1, -1), beta0.reshape(1, -1))

    h0s = h0.reshape(2 * NP, 128)
    (p1,) = _sc_colblock(h0s, srcH1, dstH1)
    (p2,) = _sc_colblock(h0s, srcH2, dstH2)
    z1, st1 = _dense1(h0, p1, p2, da, db, W_self1, W_neigh1,
                      b1.reshape(1, -1))
    (h1,) = _norm(z1, st1, gamma1.reshape(1, -1), beta1.reshape(1, -1))

    Wn2p = jnp.pad(W_neigh2, ((0, 0), (0, 88)))
    Ws2p = jnp.pad(W_self2, ((0, 0), (0, 88)))
    b2p = jnp.pad(b2, (0, 88)).reshape(1, -1)
    pr2, t2 = _proj2(h1, Wn2p, Ws2p, b2p)
    (q2,) = _sc_edge64(pr2, srcES, dstES)
    (o,) = _final(t2, q2, da, db)
    return o[:N, :40]
